# Initial kernel scaffold; baseline (speedup 1.0000x reference)
import jax
import jax.numpy as jnp
from jax.experimental import pallas as pl


def kernel(state, action, bs, W_conv, b_conv, ln_gamma, ln_beta, W2, b2, W3, b3):
    raise NotImplementedError("write your pallas kernel here")


# speedup vs baseline: 43.8917x; 43.8917x over previous
import functools

import jax
import jax.numpy as jnp
from jax.experimental import pallas as pl
from jax.experimental.pallas import tpu as pltpu

_INV_SQRT2 = 0.7071067811865476
_INV_SQRT3 = 0.5773502691896258


def _fused_body(saE_ref, W_ref, bconv_ref, gamma_ref, beta_ref,
                W2_ref, b2_ref, W3_ref, b3_ref, out_ref, acc_ref,
                *, C, Ns, M):
    i = pl.program_id(0)
    K = pl.num_programs(0)

    @pl.when(i == 0)
    def _init():
        acc_ref[...] = jnp.zeros_like(acc_ref)

    s = i * C
    prev = saE_ref[pl.ds(s, C), :]
    cent = saE_ref[pl.ds(s + 1, C), :]
    nxt = saE_ref[pl.ds(s + 2, C), :]

    g = s + jax.lax.broadcasted_iota(jnp.int32, (C, 1), 0)
    gm = g % Ns
    end_c = (gm == 0) | (gm == Ns - 1)
    d_c = jnp.where(end_c, _INV_SQRT2, _INV_SQRT3)
    gp = gm - 1
    d_p = jnp.where(gm == 0, 0.0,
                    jnp.where((gp == 0) | (gp == Ns - 1), _INV_SQRT2, _INV_SQRT3))
    gn = gm + 1
    d_n = jnp.where(gm == Ns - 1, 0.0,
                    jnp.where((gn == 0) | (gn == Ns - 1), _INV_SQRT2, _INV_SQRT3))

    v = d_c * (d_p * prev + d_c * cent + d_n * nxt)

    h = jnp.dot(v, W_ref[...], preferred_element_type=jnp.float32, precision=jax.lax.Precision.HIGHEST)
    h = jnp.maximum(h + bconv_ref[...], 0.0)

    mu = jnp.mean(h, axis=1, keepdims=True)
    hc = h - mu
    var = jnp.mean(hc * hc, axis=1, keepdims=True)
    rstd = 1.0 / jnp.sqrt(var + 1e-5)
    rstd = jnp.where(g < M, rstd, 0.0)
    acc_ref[...] += jnp.sum(hc * rstd, axis=0, keepdims=True)

    @pl.when(i == K - 1)
    def _finish():
        pooled = acc_ref[...] * gamma_ref[...] + jnp.float32(M) * beta_ref[...]
        z = jnp.dot(pooled, W2_ref[...], preferred_element_type=jnp.float32, precision=jax.lax.Precision.HIGHEST)
        z = jnp.maximum(z + b2_ref[...], 0.0)
        out_ref[...] = (jnp.dot(z, W3_ref[...], preferred_element_type=jnp.float32, precision=jax.lax.Precision.HIGHEST)
                        + b3_ref[...])


def kernel(state, action, bs, W_conv, b_conv, ln_gamma, ln_beta, W2, b2, W3, b3):
    B, Ns = state.shape[0], state.shape[1]
    M = B * Ns
    H = W_conv.shape[1]
    F = 8
    C = 2048
    K = -(-M // C)

    sa = jnp.concatenate([state, action], axis=-1).reshape(M, 5)
    saE = jnp.pad(sa, ((1, 1 + K * C - M), (0, F - 5)))
    WP = jnp.pad(W_conv, ((0, F - 5), (0, 0)))

    body = functools.partial(_fused_body, C=C, Ns=Ns, M=M)
    full = lambda a: pl.BlockSpec(a.shape, lambda i: (0,) * a.ndim)
    args = (saE, WP, b_conv.reshape(1, H), ln_gamma.reshape(1, H),
            ln_beta.reshape(1, H), W2, b2.reshape(1, H), W3,
            b3.reshape(1, 1))
    out = pl.pallas_call(
        body,
        grid=(K,),
        in_specs=[full(a) for a in args],
        out_specs=pl.BlockSpec((1, 1), lambda i: (0, 0)),
        out_shape=jax.ShapeDtypeStruct((1, 1), jnp.float32),
        scratch_shapes=[pltpu.VMEM((1, H), jnp.float32)],
    )(*args)
    return out


# speedup vs baseline: 117.6428x; 2.6803x over previous
import functools

import jax
import jax.numpy as jnp
from jax.experimental import pallas as pl
from jax.experimental.pallas import tpu as pltpu

_INV_SQRT2 = 0.7071067811865476
_INV_SQRT3 = 0.5773502691896258
_HIGH = jax.lax.Precision.HIGHEST


def _dotg(a, b, dims):
    return jax.lax.dot_general(a, b, (dims, ((), ())),
                               preferred_element_type=jnp.float32,
                               precision=_HIGH)


def _fused_body(sT_ref, aT_ref, Ws_ref, Wa_ref, bconv_ref, gamma_ref,
                beta_ref, W2_ref, b2_ref, W3_ref, b3_ref, out_ref,
                acc_ref, c0_ref, *, C, Ns, M, H):
    i = pl.program_id(0)
    K = pl.num_programs(0)

    @pl.when(i == 0)
    def _init():
        acc_ref[...] = jnp.zeros_like(acc_ref)
        c0_ref[...] = jnp.zeros_like(c0_ref)

    s = i * C
    swide = sT_ref[:, pl.ds(s, C + 128)]
    awide = aT_ref[:, pl.ds(s, C + 128)]
    sp, sc, sn = swide[:, :C], swide[:, 1:C + 1], swide[:, 2:C + 2]
    ap, ac, an = awide[:, :C], awide[:, 1:C + 1], awide[:, 2:C + 2]

    g = s + jax.lax.broadcasted_iota(jnp.int32, (1, C), 1)
    gm = g if Ns == M else g % Ns
    d_c = jnp.where((gm == 0) | (gm == Ns - 1), _INV_SQRT2, _INV_SQRT3)
    d_p = jnp.where(gm == 0, 0.0,
                    jnp.where((gm == 1) | (gm == Ns), _INV_SQRT2, _INV_SQRT3))
    d_n = jnp.where(gm == Ns - 1, 0.0,
                    jnp.where((gm == Ns - 2) | (gm == -1), _INV_SQRT2, _INV_SQRT3))

    vs = d_c * (d_p * sp + d_c * sc + d_n * sn)
    va = d_c * (d_p * ap + d_c * ac + d_n * an)

    hT = (_dotg(Ws_ref[...], vs, ((0,), (0,))) +
          _dotg(Wa_ref[...], va, ((0,), (0,))) + bconv_ref[...])
    hT = jnp.maximum(hT, 0.0)

    s1 = jnp.sum(hT, axis=0, keepdims=True)
    s2 = jnp.sum(hT * hT, axis=0, keepdims=True)
    mu = s1 * (1.0 / H)
    var = s2 * (1.0 / H) - mu * mu
    rstd = 1.0 / jnp.sqrt(var + 1e-5)
    rstd = jnp.where(g < M, rstd, 0.0)

    acc_ref[...] += _dotg(hT, rstd, ((1,), (1,)))
    c0_ref[...] += _dotg(rstd, mu, ((1,), (1,)))

    @pl.when(i == K - 1)
    def _finish():
        pooled = gamma_ref[...] * (acc_ref[...] - c0_ref[...]) \
            + jnp.float32(M) * beta_ref[...]
        z = _dotg(W2_ref[...], pooled, ((0,), (0,))) + b2_ref[...]
        z = jnp.maximum(z, 0.0)
        out_ref[...] = _dotg(W3_ref[...], z, ((0,), (0,))) + b3_ref[...]


def kernel(state, action, bs, W_conv, b_conv, ln_gamma, ln_beta, W2, b2, W3, b3):
    B, Ns = state.shape[0], state.shape[1]
    M = B * Ns
    H = W_conv.shape[1]
    C = 4096
    K = -(-M // C)
    L = K * C + 128

    sT = jnp.pad(state.reshape(M, 3).T, ((0, 0), (1, L - M - 1)))
    aT = jnp.pad(action.reshape(M, 2).T, ((0, 0), (1, L - M - 1)))

    body = functools.partial(_fused_body, C=C, Ns=Ns, M=M, H=H)
    full = lambda a: pl.BlockSpec(a.shape, lambda i: (0,) * a.ndim)
    args = (sT, aT, W_conv[0:3], W_conv[3:5], b_conv.reshape(H, 1),
            ln_gamma.reshape(H, 1), ln_beta.reshape(H, 1), W2,
            b2.reshape(H, 1), W3, b3.reshape(1, 1))
    out = pl.pallas_call(
        body,
        grid=(K,),
        in_specs=[full(a) for a in args],
        out_specs=pl.BlockSpec((1, 1), lambda i: (0, 0)),
        out_shape=jax.ShapeDtypeStruct((1, 1), jnp.float32),
        scratch_shapes=[pltpu.VMEM((H, 1), jnp.float32),
                        pltpu.VMEM((1, 1), jnp.float32)],
    )(*args)
    return out


# speedup vs baseline: 148.2453x; 1.2601x over previous
import functools

import jax
import jax.numpy as jnp
from jax.experimental import pallas as pl
from jax.experimental.pallas import tpu as pltpu

_INV_SQRT2 = 0.7071067811865476
_INV_SQRT3 = 0.5773502691896258


def _dotg(a, b, dims, prec):
    return jax.lax.dot_general(a, b, (dims, ((), ())),
                               preferred_element_type=jnp.float32,
                               precision=prec)


def _fused_body(sT_ref, aT_ref, Ws_ref, Wa_ref, bconv_ref, gamma_ref,
                beta_ref, W2_ref, b2_ref, W3_ref, b3_ref, out_ref,
                acc_ref, *, C, Ns, M, H):
    i = pl.program_id(0)
    K = pl.num_programs(0)

    @pl.when(i == 0)
    def _init():
        acc_ref[...] = jnp.zeros_like(acc_ref)

    s = i * C
    swide = sT_ref[:, pl.ds(s, C + 128)]
    awide = aT_ref[:, pl.ds(s, C + 128)]

    xwT = (_dotg(Ws_ref[...], swide, ((0,), (0,)), None) +
           _dotg(Wa_ref[...], awide, ((0,), (0,)), None))
    xp, xc, xn = xwT[:, :C], xwT[:, 1:C + 1], xwT[:, 2:C + 2]

    g = s + jax.lax.broadcasted_iota(jnp.int32, (1, C), 1)
    gm = g if Ns == M else g % Ns
    d_c = jnp.where((gm == 0) | (gm == Ns - 1), _INV_SQRT2, _INV_SQRT3)
    d_p = jnp.where(gm == 0, 0.0,
                    jnp.where((gm == 1) | (gm == Ns), _INV_SQRT2, _INV_SQRT3))
    d_n = jnp.where(gm == Ns - 1, 0.0,
                    jnp.where((gm == Ns - 2) | (gm == -1), _INV_SQRT2, _INV_SQRT3))

    aggT = d_c * (d_p * xp + d_c * xc + d_n * xn) + bconv_ref[...]
    hT = jnp.maximum(aggT, 0.0)

    mu = jnp.sum(hT, axis=0, keepdims=True) * (1.0 / H)
    hcT = hT - mu
    var = jnp.sum(hcT * hcT, axis=0, keepdims=True) * (1.0 / H)
    rstd = 1.0 / jnp.sqrt(var + 1e-5)
    rstd = jnp.where(g < M, rstd, 0.0)

    acc_ref[...] += _dotg(rstd, hcT, ((1,), (1,)),
                          jax.lax.Precision.HIGHEST)

    @pl.when(i == K - 1)
    def _finish():
        pooled = gamma_ref[...] * acc_ref[...] \
            + jnp.float32(M) * beta_ref[...]
        z = _dotg(pooled, W2_ref[...], ((1,), (0,)), None) + b2_ref[...]
        z = jnp.maximum(z, 0.0)
        out_ref[...] = _dotg(z, W3_ref[...], ((1,), (0,)), None) + b3_ref[...]


def kernel(state, action, bs, W_conv, b_conv, ln_gamma, ln_beta, W2, b2, W3, b3):
    B, Ns = state.shape[0], state.shape[1]
    M = B * Ns
    H = W_conv.shape[1]
    C = 4096
    K = -(-M // C)
    L = K * C + 128

    sT = jnp.pad(state.reshape(M, 3).T, ((0, 0), (1, L - M - 1)))
    aT = jnp.pad(action.reshape(M, 2).T, ((0, 0), (1, L - M - 1)))

    body = functools.partial(_fused_body, C=C, Ns=Ns, M=M, H=H)
    full = lambda a: pl.BlockSpec(a.shape, lambda i: (0,) * a.ndim)
    args = (sT, aT, W_conv[0:3], W_conv[3:5], b_conv.reshape(H, 1),
            ln_gamma.reshape(1, H), ln_beta.reshape(1, H), W2,
            b2.reshape(1, H), W3, b3.reshape(1, 1))
    out = pl.pallas_call(
        body,
        grid=(K,),
        in_specs=[full(a) for a in args],
        out_specs=pl.BlockSpec((1, 1), lambda i: (0, 0)),
        out_shape=jax.ShapeDtypeStruct((1, 1), jnp.float32),
        scratch_shapes=[pltpu.VMEM((1, H), jnp.float32)],
    )(*args)
    return out


# speedup vs baseline: 172.1707x; 1.1614x over previous
import functools

import jax
import jax.numpy as jnp
from jax.experimental import pallas as pl
from jax.experimental.pallas import tpu as pltpu

_INV_SQRT2 = 0.7071067811865476
_INV_SQRT3 = 0.5773502691896258


def _dotg(a, b, dims, prec):
    return jax.lax.dot_general(a, b, (dims, ((), ())),
                               preferred_element_type=jnp.float32,
                               precision=prec)


def _fused_body(saT_ref, Wc_ref, bconv_ref, gamma_ref,
                beta_ref, W2_ref, b2_ref, W3_ref, b3_ref, out_ref,
                acc_ref, *, C, Ns, M, H):
    i = pl.program_id(0)
    K = pl.num_programs(0)

    @pl.when(i == 0)
    def _init():
        acc_ref[...] = jnp.zeros_like(acc_ref)

    s = i * C
    swide = saT_ref[:, pl.ds(s, C + 128)]

    xwT = _dotg(Wc_ref[...], swide, ((0,), (0,)), None)
    xp, xc, xn = xwT[:, :C], xwT[:, 1:C + 1], xwT[:, 2:C + 2]

    g = s + jax.lax.broadcasted_iota(jnp.int32, (1, C), 1)
    gm = g if Ns == M else g % Ns
    d_c = jnp.where((gm == 0) | (gm == Ns - 1), _INV_SQRT2, _INV_SQRT3)
    d_p = jnp.where(gm == 0, 0.0,
                    jnp.where((gm == 1) | (gm == Ns), _INV_SQRT2, _INV_SQRT3))
    d_n = jnp.where(gm == Ns - 1, 0.0,
                    jnp.where((gm == Ns - 2) | (gm == -1), _INV_SQRT2, _INV_SQRT3))

    aggT = d_c * (d_p * xp + d_c * xc + d_n * xn) + bconv_ref[...]
    hT = jnp.maximum(aggT, 0.0)

    mu = jnp.sum(hT, axis=0, keepdims=True) * (1.0 / H)
    hcT = hT - mu
    var = jnp.sum(hcT * hcT, axis=0, keepdims=True) * (1.0 / H)
    rstd = 1.0 / jnp.sqrt(var + 1e-5)
    rstd = jnp.where(g < M, rstd, 0.0)

    acc_ref[...] += _dotg(rstd, hcT, ((1,), (1,)),
                          jax.lax.Precision.HIGHEST)

    @pl.when(i == K - 1)
    def _finish():
        pooled = gamma_ref[...] * acc_ref[...] \
            + jnp.float32(M) * beta_ref[...]
        z = _dotg(pooled, W2_ref[...], ((1,), (0,)), None) + b2_ref[...]
        z = jnp.maximum(z, 0.0)
        out_ref[...] = _dotg(z, W3_ref[...], ((1,), (0,)), None) + b3_ref[...]


def kernel(state, action, bs, W_conv, b_conv, ln_gamma, ln_beta, W2, b2, W3, b3):
    B, Ns = state.shape[0], state.shape[1]
    M = B * Ns
    H = W_conv.shape[1]
    C = 6400
    K = -(-M // C)
    L = K * C + 128

    saT = jnp.pad(jnp.concatenate([state.reshape(M, 3).T,
                                   action.reshape(M, 2).T], axis=0),
                  ((0, 0), (1, L - M - 1)))

    body = functools.partial(_fused_body, C=C, Ns=Ns, M=M, H=H)
    full = lambda a: pl.BlockSpec(a.shape, lambda i: (0,) * a.ndim)
    args = (saT, W_conv, b_conv.reshape(H, 1),
            ln_gamma.reshape(1, H), ln_beta.reshape(1, H), W2,
            b2.reshape(1, H), W3, b3.reshape(1, 1))
    out = pl.pallas_call(
        body,
        grid=(K,),
        in_specs=[full(a) for a in args],
        out_specs=pl.BlockSpec((1, 1), lambda i: (0, 0)),
        out_shape=jax.ShapeDtypeStruct((1, 1), jnp.float32),
        scratch_shapes=[pltpu.VMEM((1, H), jnp.float32)],
    )(*args)
    return out
